# baseline (speedup 1.0000x reference)
import functools

import jax
import jax.numpy as jnp
from jax import lax
from jax.experimental import pallas as pl
from jax.experimental.pallas import tpu as pltpu
from jax.experimental.pallas import tpu_sc as plsc

_BM = 512
_KN = 2048
_SC_CORES = 2
_SC_SUBCORES = 16


def _enc_argmin_call(x, W1, b1, W2, b2, codebook):
    B, Din = x.shape
    H = W1.shape[1]
    K, D = codebook.shape
    NB = B // _BM
    NKC = K // _KN

    def body(x_ref, W1_ref, b1_ref, W2_ref, b2_ref, C_ref, z_ref, idx_ref,
             cn_ref):
        i = pl.program_id(0)

        @pl.when(i == 0)
        def _():
            c = C_ref[...]
            cn_ref[0, :] = jnp.sum(c * c, axis=1)

        xb = x_ref[...]
        h = jnp.maximum(
            jnp.dot(xb, W1_ref[...], precision=lax.Precision.DEFAULT)
            + b1_ref[...], 0.0)
        z = jnp.dot(h, W2_ref[...],
                    precision=lax.Precision.DEFAULT) + b2_ref[...]
        z_ref[...] = z
        s = jnp.sum(z * z, axis=1, keepdims=True)

        best_v = jnp.full((_BM,), jnp.inf, jnp.float32)
        best_i = jnp.zeros((_BM,), jnp.int32)
        for kc in range(NKC):
            Cc = C_ref[kc * _KN:(kc + 1) * _KN, :]
            m = lax.dot_general(z, Cc, (((1,), (1,)), ((), ())),
                                precision=lax.Precision.DEFAULT)
            d = s - 2.0 * m + cn_ref[0, kc * _KN:(kc + 1) * _KN][None, :]
            dmin = jnp.min(d, axis=1)
            gi = lax.broadcasted_iota(jnp.int32, (_BM, _KN), 1) + kc * _KN
            li = jnp.min(
                jnp.where(d == dmin[:, None], gi, jnp.iinfo(jnp.int32).max),
                axis=1)
            upd = dmin < best_v
            best_i = jnp.where(upd, li, best_i)
            best_v = jnp.minimum(best_v, dmin)
        idx_ref[0, 0, :] = best_i

    return pl.pallas_call(
        body,
        grid=(NB,),
        in_specs=[
            pl.BlockSpec((_BM, Din), lambda i: (i, 0)),
            pl.BlockSpec((Din, H), lambda i: (0, 0)),
            pl.BlockSpec((1, H), lambda i: (0, 0)),
            pl.BlockSpec((H, D), lambda i: (0, 0)),
            pl.BlockSpec((1, D), lambda i: (0, 0)),
            pl.BlockSpec((K, D), lambda i: (0, 0)),
        ],
        out_specs=[
            pl.BlockSpec((_BM, D), lambda i: (i, 0)),
            pl.BlockSpec((1, 1, _BM), lambda i: (i, 0, 0)),
        ],
        out_shape=[
            jax.ShapeDtypeStruct((B, D), jnp.float32),
            jax.ShapeDtypeStruct((NB, 1, _BM), jnp.int32),
        ],
        scratch_shapes=[pltpu.VMEM((1, K), jnp.float32)],
    )(x, W1, b1, W2, b2, codebook)


def _sc_gather(table, idx):
    K, D = table.shape
    B = idx.shape[0]
    nw = _SC_CORES * _SC_SUBCORES
    b_per_w = B // nw
    mesh = plsc.VectorSubcoreMesh(core_axis_name="c", subcore_axis_name="s")

    @functools.partial(
        pl.kernel,
        mesh=mesh,
        out_type=jax.ShapeDtypeStruct((B, D), jnp.float32),
        scratch_types=[
            pltpu.VMEM((b_per_w,), jnp.int32),
            pltpu.VMEM((b_per_w, D), jnp.float32),
            pltpu.SemaphoreType.DMA,
        ],
    )
    def gather(table_hbm, idx_hbm, out_hbm, idx_v, rows_v, sem):
        wid = lax.axis_index("s") * _SC_CORES + lax.axis_index("c")
        base = wid * b_per_w
        pltpu.sync_copy(idx_hbm.at[pl.ds(base, b_per_w)], idx_v)
        pltpu.async_copy(table_hbm.at[idx_v], rows_v, sem).wait()
        pltpu.sync_copy(rows_v, out_hbm.at[pl.ds(base, b_per_w)])

    return gather(table, idx)


def _decoder_call(z, zq, Wd1, bd1, Wd2, bd2):
    B, D = z.shape
    H = Wd1.shape[1]
    Dout = Wd2.shape[1]
    NB = B // _BM

    def body(z_ref, zq_ref, Wd1_ref, bd1_ref, Wd2_ref, bd2_ref, recon_ref,
             loss_ref, acc_ref):
        i = pl.program_id(0)
        z = z_ref[...]
        zq = zq_ref[...]
        zq_st = z + (zq - z)
        hd = jnp.maximum(
            jnp.dot(zq_st.astype(jnp.bfloat16), Wd1_ref[...],
                    preferred_element_type=jnp.float32) + bd1_ref[...], 0.0)
        y = jnp.dot(hd.astype(jnp.bfloat16), Wd2_ref[...],
                    preferred_element_type=jnp.float32) + bd2_ref[...]
        recon_ref[...] = jax.nn.sigmoid(y)
        part = jnp.sum((zq - z) ** 2)
        acc = jnp.where(i == 0, 0.0, acc_ref[0]) + part
        acc_ref[0] = acc

        @pl.when(i == NB - 1)
        def _():
            loss_ref[...] = jnp.full((1, 1), acc, jnp.float32)

    return pl.pallas_call(
        body,
        grid=(NB,),
        in_specs=[
            pl.BlockSpec((_BM, D), lambda i: (i, 0)),
            pl.BlockSpec((_BM, D), lambda i: (i, 0)),
            pl.BlockSpec((D, H), lambda i: (0, 0)),
            pl.BlockSpec((1, H), lambda i: (0, 0)),
            pl.BlockSpec((H, Dout), lambda i: (0, 0)),
            pl.BlockSpec((1, Dout), lambda i: (0, 0)),
        ],
        out_specs=[
            pl.BlockSpec((_BM, Dout), lambda i: (i, 0)),
            pl.BlockSpec((1, 1), lambda i: (0, 0)),
        ],
        out_shape=[
            jax.ShapeDtypeStruct((B, Dout), jnp.float32),
            jax.ShapeDtypeStruct((1, 1), jnp.float32),
        ],
        scratch_shapes=[pltpu.SMEM((1,), jnp.float32)],
    )(z, zq, Wd1, bd1, Wd2, bd2)


_NSPLIT = 2


def kernel(x, W1, b1, W2, b2, codebook, Wd1, bd1, Wd2, bd2):
    B = x.shape[0]
    BC = B // _NSPLIT
    b1r, b2r = b1.reshape(1, -1), b2.reshape(1, -1)
    bd1r, bd2r = bd1.reshape(1, -1), bd2.reshape(1, -1)
    Wd1b = Wd1.astype(jnp.bfloat16)
    Wd2b = Wd2.astype(jnp.bfloat16)
    recons, idxs, sums = [], [], []
    for c in range(_NSPLIT):
        xc = lax.slice_in_dim(x, c * BC, (c + 1) * BC, axis=0)
        z, idx3 = _enc_argmin_call(xc, W1, b1r, W2, b2r, codebook)
        indices = idx3.reshape(BC)
        zq = _sc_gather(codebook, indices)
        recon, sum11 = _decoder_call(z, zq, Wd1b, bd1r, Wd2b, bd2r)
        recons.append(recon)
        idxs.append(indices)
        sums.append(sum11.reshape(()))
    total = sums[0]
    for s in sums[1:]:
        total = total + s
    mean = total / (B * codebook.shape[1])
    loss = mean + 0.25 * mean
    return (jnp.concatenate(recons, axis=0), loss,
            jnp.concatenate(idxs, axis=0))


# speedup vs baseline: 1.0815x; 1.0815x over previous
import functools

import jax
import jax.numpy as jnp
from jax import lax
from jax.experimental import pallas as pl
from jax.experimental.pallas import tpu as pltpu
from jax.experimental.pallas import tpu_sc as plsc

_BM = 512
_KN = 2048
_SC_CORES = 2
_SC_SUBCORES = 16


def _enc_argmin_call(x, W1, b1, W2, b2, codebook):
    B, Din = x.shape
    H = W1.shape[1]
    K, D = codebook.shape
    NB = B // _BM
    NKC = K // _KN

    def body(x_ref, W1_ref, b1_ref, W2_ref, b2_ref, C_ref, z_ref, idx_ref,
             cn_ref):
        i = pl.program_id(0)

        @pl.when(i == 0)
        def _():
            c = C_ref[...]
            cn_ref[0, :] = jnp.sum(c * c, axis=1)

        xb = x_ref[...]
        h = jnp.maximum(
            jnp.dot(xb, W1_ref[...], precision=lax.Precision.DEFAULT)
            + b1_ref[...], 0.0)
        z = jnp.dot(h, W2_ref[...],
                    precision=lax.Precision.DEFAULT) + b2_ref[...]
        z_ref[...] = z
        s = jnp.sum(z * z, axis=1, keepdims=True)

        best_v = jnp.full((_BM,), jnp.inf, jnp.float32)
        best_i = jnp.zeros((_BM,), jnp.int32)
        for kc in range(NKC):
            Cc = C_ref[kc * _KN:(kc + 1) * _KN, :]
            m = lax.dot_general(z, Cc, (((1,), (1,)), ((), ())),
                                precision=lax.Precision.DEFAULT)
            d = s - 2.0 * m + cn_ref[0, kc * _KN:(kc + 1) * _KN][None, :]
            dmin = jnp.min(d, axis=1)
            gi = lax.broadcasted_iota(jnp.int32, (_BM, _KN), 1) + kc * _KN
            li = jnp.min(
                jnp.where(d == dmin[:, None], gi, jnp.iinfo(jnp.int32).max),
                axis=1)
            upd = dmin < best_v
            best_i = jnp.where(upd, li, best_i)
            best_v = jnp.minimum(best_v, dmin)
        idx_ref[0, 0, :] = best_i

    return pl.pallas_call(
        body,
        grid=(NB,),
        in_specs=[
            pl.BlockSpec((_BM, Din), lambda i: (i, 0)),
            pl.BlockSpec((Din, H), lambda i: (0, 0)),
            pl.BlockSpec((1, H), lambda i: (0, 0)),
            pl.BlockSpec((H, D), lambda i: (0, 0)),
            pl.BlockSpec((1, D), lambda i: (0, 0)),
            pl.BlockSpec((K, D), lambda i: (0, 0)),
        ],
        out_specs=[
            pl.BlockSpec((_BM, D), lambda i: (i, 0)),
            pl.BlockSpec((1, 1, _BM), lambda i: (i, 0, 0)),
        ],
        out_shape=[
            jax.ShapeDtypeStruct((B, D), jnp.float32),
            jax.ShapeDtypeStruct((NB, 1, _BM), jnp.int32),
        ],
        scratch_shapes=[pltpu.VMEM((1, K), jnp.float32)],
    )(x, W1, b1, W2, b2, codebook)


def _sc_gather(table, idx):
    K, D = table.shape
    B = idx.shape[0]
    nw = _SC_CORES * _SC_SUBCORES
    b_per_w = B // nw
    mesh = plsc.VectorSubcoreMesh(core_axis_name="c", subcore_axis_name="s")

    nch = 4
    bch = b_per_w // nch

    @functools.partial(
        pl.kernel,
        mesh=mesh,
        out_type=jax.ShapeDtypeStruct((B, D), jnp.float32),
        scratch_types=[
            pltpu.VMEM((b_per_w,), jnp.int32),
            pltpu.VMEM((bch, D), jnp.float32),
            pltpu.VMEM((bch, D), jnp.float32),
            pltpu.SemaphoreType.DMA,
            pltpu.SemaphoreType.DMA,
            pltpu.SemaphoreType.DMA,
            pltpu.SemaphoreType.DMA,
        ],
    )
    def gather(table_hbm, idx_hbm, out_hbm, idx_v, rows0, rows1, g0, g1,
               w0, w1):
        wid = lax.axis_index("s") * _SC_CORES + lax.axis_index("c")
        base = wid * b_per_w
        pltpu.sync_copy(idx_hbm.at[pl.ds(base, b_per_w)], idx_v)
        bufs = (rows0, rows1)
        gsem = (g0, g1)
        wsem = (w0, w1)
        wcopies = [None, None]
        for ch in range(nch):
            p = ch % 2
            if wcopies[p] is not None:
                wcopies[p].wait()
            gcopy = pltpu.async_copy(
                table_hbm.at[idx_v.at[pl.ds(ch * bch, bch)]], bufs[p],
                gsem[p])
            gcopy.wait()
            wcopies[p] = pltpu.async_copy(
                bufs[p], out_hbm.at[pl.ds(base + ch * bch, bch)], wsem[p])
        wcopies[0].wait()
        wcopies[1].wait()

    return gather(table, idx)


def _decoder_call(z, zq, Wd1, bd1, Wd2, bd2):
    B, D = z.shape
    H = Wd1.shape[1]
    Dout = Wd2.shape[1]
    NB = B // _BM

    def body(z_ref, zq_ref, Wd1_ref, bd1_ref, Wd2_ref, bd2_ref, recon_ref,
             loss_ref, acc_ref):
        i = pl.program_id(0)
        z = z_ref[...]
        zq = zq_ref[...]
        zq_st = z + (zq - z)
        hd = jnp.maximum(
            jnp.dot(zq_st.astype(jnp.bfloat16), Wd1_ref[...],
                    preferred_element_type=jnp.float32) + bd1_ref[...], 0.0)
        y = jnp.dot(hd.astype(jnp.bfloat16), Wd2_ref[...],
                    preferred_element_type=jnp.float32) + bd2_ref[...]
        recon_ref[...] = jax.nn.sigmoid(y)
        part = jnp.sum((zq - z) ** 2)
        acc = jnp.where(i == 0, 0.0, acc_ref[0]) + part
        acc_ref[0] = acc

        @pl.when(i == NB - 1)
        def _():
            loss_ref[...] = jnp.full((1, 1), acc, jnp.float32)

    return pl.pallas_call(
        body,
        grid=(NB,),
        in_specs=[
            pl.BlockSpec((_BM, D), lambda i: (i, 0)),
            pl.BlockSpec((_BM, D), lambda i: (i, 0)),
            pl.BlockSpec((D, H), lambda i: (0, 0)),
            pl.BlockSpec((1, H), lambda i: (0, 0)),
            pl.BlockSpec((H, Dout), lambda i: (0, 0)),
            pl.BlockSpec((1, Dout), lambda i: (0, 0)),
        ],
        out_specs=[
            pl.BlockSpec((_BM, Dout), lambda i: (i, 0)),
            pl.BlockSpec((1, 1), lambda i: (0, 0)),
        ],
        out_shape=[
            jax.ShapeDtypeStruct((B, Dout), jnp.float32),
            jax.ShapeDtypeStruct((1, 1), jnp.float32),
        ],
        scratch_shapes=[pltpu.SMEM((1,), jnp.float32)],
    )(z, zq, Wd1, bd1, Wd2, bd2)


def kernel(x, W1, b1, W2, b2, codebook, Wd1, bd1, Wd2, bd2):
    B = x.shape[0]
    z, idx3 = _enc_argmin_call(x, W1, b1.reshape(1, -1), W2,
                               b2.reshape(1, -1), codebook)
    indices = idx3.reshape(B)
    zq = _sc_gather(codebook, indices)
    recon, sum11 = _decoder_call(z, zq, Wd1.astype(jnp.bfloat16),
                                 bd1.reshape(1, -1),
                                 Wd2.astype(jnp.bfloat16),
                                 bd2.reshape(1, -1))
    mean = sum11.reshape(()) / (B * codebook.shape[1])
    loss = mean + 0.25 * mean
    return recon, loss, indices


# speedup vs baseline: 1.1166x; 1.0324x over previous
import functools

import jax
import jax.numpy as jnp
from jax import lax
from jax.experimental import pallas as pl
from jax.experimental.pallas import tpu as pltpu
from jax.experimental.pallas import tpu_sc as plsc

_BM = 512
_KN = 2048
_SC_CORES = 2
_SC_SUBCORES = 16


def _enc_argmin_call(x, W1, b1, W2, b2, codebook):
    B, Din = x.shape
    H = W1.shape[1]
    K, D = codebook.shape
    NB = B // _BM
    NKC = K // _KN

    def body(x_ref, W1_ref, b1_ref, W2_ref, b2_ref, C_ref, z_ref, idx_ref,
             cn_ref, cm2_ref, iota_ref):
        i = pl.program_id(0)

        @pl.when(i == 0)
        def _():
            c = C_ref[...]
            cn_ref[0, :] = jnp.sum(c * c, axis=1)
            cm2_ref[...] = c * -2.0
            iota_ref[...] = lax.broadcasted_iota(
                jnp.int32, (1, K), 1).astype(jnp.float32)

        xb = x_ref[...]
        h = jnp.maximum(
            jnp.dot(xb, W1_ref[...], precision=lax.Precision.DEFAULT)
            + b1_ref[...], 0.0)
        z = jnp.dot(h, W2_ref[...],
                    precision=lax.Precision.DEFAULT) + b2_ref[...]
        z_ref[...] = z
        s = lax.dot_general(z * z, jnp.ones((D, 1), jnp.float32),
                            (((1,), (0,)), ((), ())),
                            precision=lax.Precision.HIGHEST)

        best_v = jnp.full((_BM,), jnp.inf, jnp.float32)
        best_i = jnp.full((_BM,), jnp.inf, jnp.float32)
        for kc in range(NKC):
            Cc = cm2_ref[kc * _KN:(kc + 1) * _KN, :]
            m2 = lax.dot_general(z, Cc, (((1,), (1,)), ((), ())),
                                 precision=lax.Precision.DEFAULT)
            d = (s + m2) + cn_ref[0, kc * _KN:(kc + 1) * _KN][None, :]
            dmin = jnp.min(d, axis=1)
            gi = iota_ref[0, kc * _KN:(kc + 1) * _KN][None, :]
            li = jnp.min(jnp.where(d == dmin[:, None], gi, jnp.inf), axis=1)
            upd = dmin < best_v
            best_i = jnp.where(upd, li, best_i)
            best_v = jnp.minimum(best_v, dmin)
        idx_ref[0, 0, :] = best_i.astype(jnp.int32)

    return pl.pallas_call(
        body,
        grid=(NB,),
        in_specs=[
            pl.BlockSpec((_BM, Din), lambda i: (i, 0)),
            pl.BlockSpec((Din, H), lambda i: (0, 0)),
            pl.BlockSpec((1, H), lambda i: (0, 0)),
            pl.BlockSpec((H, D), lambda i: (0, 0)),
            pl.BlockSpec((1, D), lambda i: (0, 0)),
            pl.BlockSpec((K, D), lambda i: (0, 0)),
        ],
        out_specs=[
            pl.BlockSpec((_BM, D), lambda i: (i, 0)),
            pl.BlockSpec((1, 1, _BM), lambda i: (i, 0, 0)),
        ],
        out_shape=[
            jax.ShapeDtypeStruct((B, D), jnp.float32),
            jax.ShapeDtypeStruct((NB, 1, _BM), jnp.int32),
        ],
        scratch_shapes=[pltpu.VMEM((1, K), jnp.float32),
                        pltpu.VMEM((K, D), jnp.float32),
                        pltpu.VMEM((1, K), jnp.float32)],
    )(x, W1, b1, W2, b2, codebook)


def _sc_gather(table, idx):
    K, D = table.shape
    B = idx.shape[0]
    nw = _SC_CORES * _SC_SUBCORES
    b_per_w = B // nw
    mesh = plsc.VectorSubcoreMesh(core_axis_name="c", subcore_axis_name="s")

    nch = 4
    bch = b_per_w // nch

    @functools.partial(
        pl.kernel,
        mesh=mesh,
        out_type=jax.ShapeDtypeStruct((B, D), jnp.float32),
        scratch_types=[
            pltpu.VMEM((b_per_w,), jnp.int32),
            pltpu.VMEM((bch, D), jnp.float32),
            pltpu.VMEM((bch, D), jnp.float32),
            pltpu.SemaphoreType.DMA,
            pltpu.SemaphoreType.DMA,
            pltpu.SemaphoreType.DMA,
            pltpu.SemaphoreType.DMA,
        ],
    )
    def gather(table_hbm, idx_hbm, out_hbm, idx_v, rows0, rows1, g0, g1,
               w0, w1):
        wid = lax.axis_index("s") * _SC_CORES + lax.axis_index("c")
        base = wid * b_per_w
        pltpu.sync_copy(idx_hbm.at[pl.ds(base, b_per_w)], idx_v)
        bufs = (rows0, rows1)
        gsem = (g0, g1)
        wsem = (w0, w1)
        wcopies = [None, None]
        for ch in range(nch):
            p = ch % 2
            if wcopies[p] is not None:
                wcopies[p].wait()
            gcopy = pltpu.async_copy(
                table_hbm.at[idx_v.at[pl.ds(ch * bch, bch)]], bufs[p],
                gsem[p])
            gcopy.wait()
            wcopies[p] = pltpu.async_copy(
                bufs[p], out_hbm.at[pl.ds(base + ch * bch, bch)], wsem[p])
        wcopies[0].wait()
        wcopies[1].wait()

    return gather(table, idx)


def _decoder_call(z, zq, Wd1, bd1, Wd2, bd2):
    B, D = z.shape
    H = Wd1.shape[1]
    Dout = Wd2.shape[1]
    NB = B // _BM

    def body(z_ref, zq_ref, Wd1_ref, bd1_ref, Wd2_ref, bd2_ref, recon_ref,
             loss_ref, acc_ref):
        i = pl.program_id(0)
        z = z_ref[...]
        zq = zq_ref[...]
        zq_st = z + (zq - z)
        hd = jnp.maximum(
            jnp.dot(zq_st.astype(jnp.bfloat16), Wd1_ref[...],
                    preferred_element_type=jnp.float32) + bd1_ref[...], 0.0)
        y = jnp.dot(hd.astype(jnp.bfloat16), Wd2_ref[...],
                    preferred_element_type=jnp.float32) + bd2_ref[...]
        recon_ref[...] = jax.nn.sigmoid(y)
        part = jnp.sum((zq - z) ** 2)
        acc = jnp.where(i == 0, 0.0, acc_ref[0]) + part
        acc_ref[0] = acc

        @pl.when(i == NB - 1)
        def _():
            loss_ref[...] = jnp.full((1, 1), acc, jnp.float32)

    return pl.pallas_call(
        body,
        grid=(NB,),
        in_specs=[
            pl.BlockSpec((_BM, D), lambda i: (i, 0)),
            pl.BlockSpec((_BM, D), lambda i: (i, 0)),
            pl.BlockSpec((D, H), lambda i: (0, 0)),
            pl.BlockSpec((1, H), lambda i: (0, 0)),
            pl.BlockSpec((H, Dout), lambda i: (0, 0)),
            pl.BlockSpec((1, Dout), lambda i: (0, 0)),
        ],
        out_specs=[
            pl.BlockSpec((_BM, Dout), lambda i: (i, 0)),
            pl.BlockSpec((1, 1), lambda i: (0, 0)),
        ],
        out_shape=[
            jax.ShapeDtypeStruct((B, Dout), jnp.float32),
            jax.ShapeDtypeStruct((1, 1), jnp.float32),
        ],
        scratch_shapes=[pltpu.SMEM((1,), jnp.float32)],
    )(z, zq, Wd1, bd1, Wd2, bd2)


def kernel(x, W1, b1, W2, b2, codebook, Wd1, bd1, Wd2, bd2):
    B = x.shape[0]
    z, idx3 = _enc_argmin_call(x, W1, b1.reshape(1, -1), W2,
                               b2.reshape(1, -1), codebook)
    indices = idx3.reshape(B)
    zq = _sc_gather(codebook, indices)
    recon, sum11 = _decoder_call(z, zq, Wd1.astype(jnp.bfloat16),
                                 bd1.reshape(1, -1),
                                 Wd2.astype(jnp.bfloat16),
                                 bd2.reshape(1, -1))
    mean = sum11.reshape(()) / (B * codebook.shape[1])
    loss = mean + 0.25 * mean
    return recon, loss, indices


# speedup vs baseline: 1.1414x; 1.0222x over previous
import functools

import jax
import jax.numpy as jnp
from jax import lax
from jax.experimental import pallas as pl
from jax.experimental.pallas import tpu as pltpu
from jax.experimental.pallas import tpu_sc as plsc

_BM = 512
_KN = 2048
_SC_CORES = 2
_SC_SUBCORES = 16


def _enc_argmin_call(x, W1, b1, W2, b2, codebook):
    B, Din = x.shape
    H = W1.shape[1]
    K, D = codebook.shape
    NB = B // _BM
    NKC = K // _KN

    def body(x_ref, W1_ref, b1_ref, W2_ref, b2_ref, C_ref, z_ref, idx_ref,
             cn_ref, cm2_ref, iota_ref):
        i = pl.program_id(0)

        @pl.when(i == 0)
        def _():
            c = C_ref[...]
            cn_ref[0, :] = jnp.sum(c * c, axis=1)
            cm2_ref[...] = c * -2.0
            iota_ref[...] = lax.broadcasted_iota(
                jnp.int32, (1, K), 1).astype(jnp.float32)

        xb = x_ref[...]
        h = jnp.maximum(
            jnp.dot(xb, W1_ref[...], precision=lax.Precision.DEFAULT)
            + b1_ref[...], 0.0)
        z = jnp.dot(h, W2_ref[...],
                    precision=lax.Precision.DEFAULT) + b2_ref[...]
        z_ref[...] = z
        s = lax.dot_general(z * z, jnp.ones((D, 1), jnp.float32),
                            (((1,), (0,)), ((), ())),
                            precision=lax.Precision.HIGHEST)

        best_v = jnp.full((_BM,), jnp.inf, jnp.float32)
        best_i = jnp.full((_BM,), jnp.inf, jnp.float32)
        for kc in range(NKC):
            Cc = cm2_ref[kc * _KN:(kc + 1) * _KN, :]
            m2 = lax.dot_general(z, Cc, (((1,), (1,)), ((), ())),
                                 precision=lax.Precision.DEFAULT)
            d = (s + m2) + cn_ref[0, kc * _KN:(kc + 1) * _KN][None, :]
            dmin = jnp.min(d, axis=1)
            gi = iota_ref[0, kc * _KN:(kc + 1) * _KN][None, :]
            li = jnp.min(jnp.where(d == dmin[:, None], gi, jnp.inf), axis=1)
            upd = dmin < best_v
            best_i = jnp.where(upd, li, best_i)
            best_v = jnp.minimum(best_v, dmin)
        idx_ref[0, 0, :] = best_i.astype(jnp.int32)

    return pl.pallas_call(
        body,
        grid=(NB,),
        in_specs=[
            pl.BlockSpec((_BM, Din), lambda i: (i, 0)),
            pl.BlockSpec((Din, H), lambda i: (0, 0)),
            pl.BlockSpec((1, H), lambda i: (0, 0)),
            pl.BlockSpec((H, D), lambda i: (0, 0)),
            pl.BlockSpec((1, D), lambda i: (0, 0)),
            pl.BlockSpec((K, D), lambda i: (0, 0)),
        ],
        out_specs=[
            pl.BlockSpec((_BM, D), lambda i: (i, 0)),
            pl.BlockSpec((1, 1, _BM), lambda i: (i, 0, 0)),
        ],
        out_shape=[
            jax.ShapeDtypeStruct((B, D), jnp.float32),
            jax.ShapeDtypeStruct((NB, 1, _BM), jnp.int32),
        ],
        scratch_shapes=[pltpu.VMEM((1, K), jnp.float32),
                        pltpu.VMEM((K, D), jnp.float32),
                        pltpu.VMEM((1, K), jnp.float32)],
    )(x, W1, b1, W2, b2, codebook)


def _sc_gather(table, idx):
    K, D = table.shape
    B = idx.shape[0]
    nw = _SC_CORES * _SC_SUBCORES
    b_per_w = B // nw
    mesh = plsc.VectorSubcoreMesh(core_axis_name="c", subcore_axis_name="s")

    @functools.partial(
        pl.kernel,
        mesh=mesh,
        out_type=jax.ShapeDtypeStruct((B, D), jnp.float32),
        scratch_types=[
            pltpu.VMEM((b_per_w,), jnp.int32),
            pltpu.VMEM((b_per_w, D), jnp.float32),
            pltpu.SemaphoreType.DMA,
        ],
    )
    def gather(table_hbm, idx_hbm, out_hbm, idx_v, rows_v, sem):
        wid = lax.axis_index("s") * _SC_CORES + lax.axis_index("c")
        base = wid * b_per_w
        pltpu.sync_copy(idx_hbm.at[pl.ds(base, b_per_w)], idx_v)
        pltpu.async_copy(table_hbm.at[idx_v], rows_v, sem).wait()
        pltpu.sync_copy(rows_v, out_hbm.at[pl.ds(base, b_per_w)])

    return gather(table, idx)


def _decoder_call(z, zq, Wd1, bd1, Wd2, bd2):
    B, D = z.shape
    H = Wd1.shape[1]
    Dout = Wd2.shape[1]
    NB = B // _BM

    def body(z_ref, zq_ref, Wd1_ref, bd1_ref, Wd2_ref, bd2_ref, recon_ref,
             loss_ref, acc_ref, w1b_ref, w2b_ref):
        i = pl.program_id(0)

        @pl.when(i == 0)
        def _():
            w1b_ref[...] = Wd1_ref[...].astype(jnp.bfloat16)
            w2b_ref[...] = Wd2_ref[...].astype(jnp.bfloat16)

        z = z_ref[...]
        zq = zq_ref[...]
        zq_st = z + (zq - z)
        hd = jnp.maximum(
            jnp.dot(zq_st.astype(jnp.bfloat16), w1b_ref[...],
                    preferred_element_type=jnp.float32) + bd1_ref[...], 0.0)
        y = jnp.dot(hd.astype(jnp.bfloat16), w2b_ref[...],
                    preferred_element_type=jnp.float32) + bd2_ref[...]
        recon_ref[...] = jax.nn.sigmoid(y)
        part = jnp.sum((zq - z) ** 2)
        acc = jnp.where(i == 0, 0.0, acc_ref[0]) + part
        acc_ref[0] = acc

        @pl.when(i == NB - 1)
        def _():
            loss_ref[...] = jnp.full((1, 1), acc, jnp.float32)

    return pl.pallas_call(
        body,
        grid=(NB,),
        in_specs=[
            pl.BlockSpec((_BM, D), lambda i: (i, 0)),
            pl.BlockSpec((_BM, D), lambda i: (i, 0)),
            pl.BlockSpec((D, H), lambda i: (0, 0)),
            pl.BlockSpec((1, H), lambda i: (0, 0)),
            pl.BlockSpec((H, Dout), lambda i: (0, 0)),
            pl.BlockSpec((1, Dout), lambda i: (0, 0)),
        ],
        out_specs=[
            pl.BlockSpec((_BM, Dout), lambda i: (i, 0)),
            pl.BlockSpec((1, 1), lambda i: (0, 0)),
        ],
        out_shape=[
            jax.ShapeDtypeStruct((B, Dout), jnp.float32),
            jax.ShapeDtypeStruct((1, 1), jnp.float32),
        ],
        scratch_shapes=[pltpu.SMEM((1,), jnp.float32),
                        pltpu.VMEM((D, H), jnp.bfloat16),
                        pltpu.VMEM((H, Dout), jnp.bfloat16)],
    )(z, zq, Wd1, bd1, Wd2, bd2)


def kernel(x, W1, b1, W2, b2, codebook, Wd1, bd1, Wd2, bd2):
    B = x.shape[0]
    z, idx3 = _enc_argmin_call(x, W1, b1.reshape(1, -1), W2,
                               b2.reshape(1, -1), codebook)
    indices = idx3.reshape(B)
    zq = _sc_gather(codebook, indices)
    recon, sum11 = _decoder_call(z, zq, Wd1, bd1.reshape(1, -1),
                                 Wd2, bd2.reshape(1, -1))
    mean = sum11.reshape(()) / (B * codebook.shape[1])
    loss = mean + 0.25 * mean
    return recon, loss, indices


# speedup vs baseline: 1.1633x; 1.0192x over previous
import functools

import jax
import jax.numpy as jnp
from jax import lax
from jax.experimental import pallas as pl
from jax.experimental.pallas import tpu as pltpu
from jax.experimental.pallas import tpu_sc as plsc

_BM = 1024
_KN = 2048
_SC_CORES = 2
_SC_SUBCORES = 16


def _enc_argmin_call(x, W1, b1, W2, b2, codebook):
    B, Din = x.shape
    H = W1.shape[1]
    K, D = codebook.shape
    NB = B // _BM
    NKC = K // _KN

    def body(x_ref, W1_ref, b1_ref, W2_ref, b2_ref, C_ref, z_ref, idx_ref,
             cn_ref, cm2_ref, iota_ref):
        i = pl.program_id(0)

        @pl.when(i == 0)
        def _():
            c = C_ref[...]
            cn_ref[0, :] = jnp.sum(c * c, axis=1)
            cm2_ref[...] = c * -2.0
            iota_ref[...] = lax.broadcasted_iota(
                jnp.int32, (1, K), 1).astype(jnp.float32)

        xb = x_ref[...]
        h = jnp.maximum(
            jnp.dot(xb, W1_ref[...], precision=lax.Precision.DEFAULT)
            + b1_ref[...], 0.0)
        z = jnp.dot(h, W2_ref[...],
                    precision=lax.Precision.DEFAULT) + b2_ref[...]
        z_ref[...] = z
        s = lax.dot_general(z * z, jnp.ones((D, 1), jnp.float32),
                            (((1,), (0,)), ((), ())),
                            precision=lax.Precision.HIGHEST)

        best_v = jnp.full((_BM,), jnp.inf, jnp.float32)
        best_i = jnp.full((_BM,), jnp.inf, jnp.float32)
        for kc in range(NKC):
            Cc = cm2_ref[kc * _KN:(kc + 1) * _KN, :]
            m2 = lax.dot_general(z, Cc, (((1,), (1,)), ((), ())),
                                 precision=lax.Precision.DEFAULT)
            d = (s + m2) + cn_ref[0, kc * _KN:(kc + 1) * _KN][None, :]
            dmin = jnp.min(d, axis=1)
            gi = iota_ref[0, kc * _KN:(kc + 1) * _KN][None, :]
            li = jnp.min(jnp.where(d == dmin[:, None], gi, jnp.inf), axis=1)
            upd = dmin < best_v
            best_i = jnp.where(upd, li, best_i)
            best_v = jnp.minimum(best_v, dmin)
        idx_ref[0, 0, :] = best_i.astype(jnp.int32)

    return pl.pallas_call(
        body,
        grid=(NB,),
        in_specs=[
            pl.BlockSpec((_BM, Din), lambda i: (i, 0)),
            pl.BlockSpec((Din, H), lambda i: (0, 0)),
            pl.BlockSpec((1, H), lambda i: (0, 0)),
            pl.BlockSpec((H, D), lambda i: (0, 0)),
            pl.BlockSpec((1, D), lambda i: (0, 0)),
            pl.BlockSpec((K, D), lambda i: (0, 0)),
        ],
        out_specs=[
            pl.BlockSpec((_BM, D), lambda i: (i, 0)),
            pl.BlockSpec((1, 1, _BM), lambda i: (i, 0, 0)),
        ],
        out_shape=[
            jax.ShapeDtypeStruct((B, D), jnp.float32),
            jax.ShapeDtypeStruct((NB, 1, _BM), jnp.int32),
        ],
        scratch_shapes=[pltpu.VMEM((1, K), jnp.float32),
                        pltpu.VMEM((K, D), jnp.float32),
                        pltpu.VMEM((1, K), jnp.float32)],
    )(x, W1, b1, W2, b2, codebook)


def _sc_gather(table, idx):
    K, D = table.shape
    B = idx.shape[0]
    nw = _SC_CORES * _SC_SUBCORES
    b_per_w = B // nw
    mesh = plsc.VectorSubcoreMesh(core_axis_name="c", subcore_axis_name="s")

    @functools.partial(
        pl.kernel,
        mesh=mesh,
        out_type=jax.ShapeDtypeStruct((B, D), jnp.float32),
        scratch_types=[
            pltpu.VMEM((b_per_w,), jnp.int32),
            pltpu.VMEM((b_per_w, D), jnp.float32),
            pltpu.SemaphoreType.DMA,
        ],
    )
    def gather(table_hbm, idx_hbm, out_hbm, idx_v, rows_v, sem):
        wid = lax.axis_index("s") * _SC_CORES + lax.axis_index("c")
        base = wid * b_per_w
        pltpu.sync_copy(idx_hbm.at[pl.ds(base, b_per_w)], idx_v)
        pltpu.async_copy(table_hbm.at[idx_v], rows_v, sem).wait()
        pltpu.sync_copy(rows_v, out_hbm.at[pl.ds(base, b_per_w)])

    return gather(table, idx)


def _decoder_call(z, zq, Wd1, bd1, Wd2, bd2):
    B, D = z.shape
    H = Wd1.shape[1]
    Dout = Wd2.shape[1]
    NB = B // _BM

    def body(z_ref, zq_ref, Wd1_ref, bd1_ref, Wd2_ref, bd2_ref, recon_ref,
             loss_ref, acc_ref, w1b_ref, w2b_ref):
        i = pl.program_id(0)

        @pl.when(i == 0)
        def _():
            w1b_ref[...] = Wd1_ref[...].astype(jnp.bfloat16)
            w2b_ref[...] = Wd2_ref[...].astype(jnp.bfloat16)

        z = z_ref[...]
        zq = zq_ref[...]
        zq_st = z + (zq - z)
        hd = jnp.maximum(
            jnp.dot(zq_st.astype(jnp.bfloat16), w1b_ref[...],
                    preferred_element_type=jnp.float32) + bd1_ref[...], 0.0)
        y = jnp.dot(hd.astype(jnp.bfloat16), w2b_ref[...],
                    preferred_element_type=jnp.float32) + bd2_ref[...]
        recon_ref[...] = jax.nn.sigmoid(y)
        part = jnp.sum((zq - z) ** 2)
        acc = jnp.where(i == 0, 0.0, acc_ref[0]) + part
        acc_ref[0] = acc

        @pl.when(i == NB - 1)
        def _():
            loss_ref[...] = jnp.full((1, 1), acc, jnp.float32)

    return pl.pallas_call(
        body,
        grid=(NB,),
        in_specs=[
            pl.BlockSpec((_BM, D), lambda i: (i, 0)),
            pl.BlockSpec((_BM, D), lambda i: (i, 0)),
            pl.BlockSpec((D, H), lambda i: (0, 0)),
            pl.BlockSpec((1, H), lambda i: (0, 0)),
            pl.BlockSpec((H, Dout), lambda i: (0, 0)),
            pl.BlockSpec((1, Dout), lambda i: (0, 0)),
        ],
        out_specs=[
            pl.BlockSpec((_BM, Dout), lambda i: (i, 0)),
            pl.BlockSpec((1, 1), lambda i: (0, 0)),
        ],
        out_shape=[
            jax.ShapeDtypeStruct((B, Dout), jnp.float32),
            jax.ShapeDtypeStruct((1, 1), jnp.float32),
        ],
        scratch_shapes=[pltpu.SMEM((1,), jnp.float32),
                        pltpu.VMEM((D, H), jnp.bfloat16),
                        pltpu.VMEM((H, Dout), jnp.bfloat16)],
    )(z, zq, Wd1, bd1, Wd2, bd2)


def kernel(x, W1, b1, W2, b2, codebook, Wd1, bd1, Wd2, bd2):
    B = x.shape[0]
    z, idx3 = _enc_argmin_call(x, W1, b1.reshape(1, -1), W2,
                               b2.reshape(1, -1), codebook)
    indices = idx3.reshape(B)
    zq = _sc_gather(codebook, indices)
    recon, sum11 = _decoder_call(z, zq, Wd1, bd1.reshape(1, -1),
                                 Wd2, bd2.reshape(1, -1))
    mean = sum11.reshape(()) / (B * codebook.shape[1])
    loss = mean + 0.25 * mean
    return recon, loss, indices


# speedup vs baseline: 1.2134x; 1.0430x over previous
import functools

import jax
import jax.numpy as jnp
from jax import lax
from jax.experimental import pallas as pl
from jax.experimental.pallas import tpu as pltpu
from jax.experimental.pallas import tpu_sc as plsc

_BM = 1024
_KN = 1024
_SC_CORES = 2
_SC_SUBCORES = 16


def _enc_argmin_call(x, W1, b1, W2, b2, codebook):
    B, Din = x.shape
    H = W1.shape[1]
    K, D = codebook.shape
    NB = B // _BM
    NKC = K // _KN

    def body(x_ref, W1_ref, b1_ref, W2_ref, b2_ref, C_ref, z_ref, idx_ref,
             cn_ref, cm2_ref, iota_ref):
        i = pl.program_id(0)

        @pl.when(i == 0)
        def _():
            c = C_ref[...]
            cn_ref[...] = jnp.sum(c * c, axis=1, keepdims=True)
            cm2_ref[...] = c * -2.0
            iota_ref[...] = lax.broadcasted_iota(
                jnp.int32, (K, 1), 0).astype(jnp.float32)

        xb = x_ref[...]
        h = jnp.maximum(
            jnp.dot(xb, W1_ref[...], precision=lax.Precision.DEFAULT)
            + b1_ref[...], 0.0)
        z = jnp.dot(h, W2_ref[...],
                    precision=lax.Precision.DEFAULT) + b2_ref[...]
        z_ref[...] = z
        s = lax.dot_general(jnp.ones((1, D), jnp.float32), z * z,
                            (((1,), (1,)), ((), ())),
                            precision=lax.Precision.HIGHEST)

        best_v = jnp.full((1, _BM), jnp.inf, jnp.float32)
        best_i = jnp.full((1, _BM), jnp.inf, jnp.float32)
        for kc in range(NKC):
            Cc = cm2_ref[kc * _KN:(kc + 1) * _KN, :]
            m2 = lax.dot_general(Cc, z, (((1,), (1,)), ((), ())),
                                 precision=lax.Precision.DEFAULT)
            d = (s + m2) + cn_ref[kc * _KN:(kc + 1) * _KN, :]
            dmin = jnp.min(d, axis=0, keepdims=True)
            gi = iota_ref[kc * _KN:(kc + 1) * _KN, :]
            li = jnp.min(jnp.where(d == dmin, gi, jnp.inf), axis=0,
                         keepdims=True)
            upd = dmin < best_v
            best_i = jnp.where(upd, li, best_i)
            best_v = jnp.minimum(best_v, dmin)
        idx_ref[0, :, :] = best_i.astype(jnp.int32)

    return pl.pallas_call(
        body,
        grid=(NB,),
        in_specs=[
            pl.BlockSpec((_BM, Din), lambda i: (i, 0)),
            pl.BlockSpec((Din, H), lambda i: (0, 0)),
            pl.BlockSpec((1, H), lambda i: (0, 0)),
            pl.BlockSpec((H, D), lambda i: (0, 0)),
            pl.BlockSpec((1, D), lambda i: (0, 0)),
            pl.BlockSpec((K, D), lambda i: (0, 0)),
        ],
        out_specs=[
            pl.BlockSpec((_BM, D), lambda i: (i, 0)),
            pl.BlockSpec((1, 1, _BM), lambda i: (i, 0, 0)),
        ],
        out_shape=[
            jax.ShapeDtypeStruct((B, D), jnp.float32),
            jax.ShapeDtypeStruct((NB, 1, _BM), jnp.int32),
        ],
        scratch_shapes=[pltpu.VMEM((K, 1), jnp.float32),
                        pltpu.VMEM((K, D), jnp.float32),
                        pltpu.VMEM((K, 1), jnp.float32)],
    )(x, W1, b1, W2, b2, codebook)


def _sc_gather(table, idx):
    K, D = table.shape
    B = idx.shape[0]
    nw = _SC_CORES * _SC_SUBCORES
    b_per_w = B // nw
    mesh = plsc.VectorSubcoreMesh(core_axis_name="c", subcore_axis_name="s")

    @functools.partial(
        pl.kernel,
        mesh=mesh,
        out_type=jax.ShapeDtypeStruct((B, D), jnp.float32),
        scratch_types=[
            pltpu.VMEM((b_per_w,), jnp.int32),
            pltpu.VMEM((b_per_w, D), jnp.float32),
            pltpu.SemaphoreType.DMA,
        ],
    )
    def gather(table_hbm, idx_hbm, out_hbm, idx_v, rows_v, sem):
        wid = lax.axis_index("s") * _SC_CORES + lax.axis_index("c")
        base = wid * b_per_w
        pltpu.sync_copy(idx_hbm.at[pl.ds(base, b_per_w)], idx_v)
        pltpu.async_copy(table_hbm.at[idx_v], rows_v, sem).wait()
        pltpu.sync_copy(rows_v, out_hbm.at[pl.ds(base, b_per_w)])

    return gather(table, idx)


def _decoder_call(z, zq, Wd1, bd1, Wd2, bd2):
    B, D = z.shape
    H = Wd1.shape[1]
    Dout = Wd2.shape[1]
    NB = B // _BM

    def body(z_ref, zq_ref, Wd1_ref, bd1_ref, Wd2_ref, bd2_ref, recon_ref,
             loss_ref, acc_ref, w1b_ref, w2b_ref):
        i = pl.program_id(0)

        @pl.when(i == 0)
        def _():
            w1b_ref[...] = Wd1_ref[...].astype(jnp.bfloat16)
            w2b_ref[...] = Wd2_ref[...].astype(jnp.bfloat16)

        z = z_ref[...]
        zq = zq_ref[...]
        zq_st = z + (zq - z)
        hd = jnp.maximum(
            jnp.dot(zq_st.astype(jnp.bfloat16), w1b_ref[...],
                    preferred_element_type=jnp.float32) + bd1_ref[...], 0.0)
        y = jnp.dot(hd.astype(jnp.bfloat16), w2b_ref[...],
                    preferred_element_type=jnp.float32) + bd2_ref[...]
        recon_ref[...] = jax.nn.sigmoid(y)
        part = jnp.sum((zq - z) ** 2)
        acc = jnp.where(i == 0, 0.0, acc_ref[0]) + part
        acc_ref[0] = acc

        @pl.when(i == NB - 1)
        def _():
            loss_ref[...] = jnp.full((1, 1), acc, jnp.float32)

    return pl.pallas_call(
        body,
        grid=(NB,),
        in_specs=[
            pl.BlockSpec((_BM, D), lambda i: (i, 0)),
            pl.BlockSpec((_BM, D), lambda i: (i, 0)),
            pl.BlockSpec((D, H), lambda i: (0, 0)),
            pl.BlockSpec((1, H), lambda i: (0, 0)),
            pl.BlockSpec((H, Dout), lambda i: (0, 0)),
            pl.BlockSpec((1, Dout), lambda i: (0, 0)),
        ],
        out_specs=[
            pl.BlockSpec((_BM, Dout), lambda i: (i, 0)),
            pl.BlockSpec((1, 1), lambda i: (0, 0)),
        ],
        out_shape=[
            jax.ShapeDtypeStruct((B, Dout), jnp.float32),
            jax.ShapeDtypeStruct((1, 1), jnp.float32),
        ],
        scratch_shapes=[pltpu.SMEM((1,), jnp.float32),
                        pltpu.VMEM((D, H), jnp.bfloat16),
                        pltpu.VMEM((H, Dout), jnp.bfloat16)],
    )(z, zq, Wd1, bd1, Wd2, bd2)


def kernel(x, W1, b1, W2, b2, codebook, Wd1, bd1, Wd2, bd2):
    B = x.shape[0]
    z, idx3 = _enc_argmin_call(x, W1, b1.reshape(1, -1), W2,
                               b2.reshape(1, -1), codebook)
    indices = idx3.reshape(B)
    zq = _sc_gather(codebook, indices)
    recon, sum11 = _decoder_call(z, zq, Wd1, bd1.reshape(1, -1),
                                 Wd2, bd2.reshape(1, -1))
    mean = sum11.reshape(()) / (B * codebook.shape[1])
    loss = mean + 0.25 * mean
    return recon, loss, indices


# speedup vs baseline: 1.2149x; 1.0012x over previous
import functools

import jax
import jax.numpy as jnp
from jax import lax
from jax.experimental import pallas as pl
from jax.experimental.pallas import tpu as pltpu
from jax.experimental.pallas import tpu_sc as plsc

_BM = 1024
_KN = 512
_SC_CORES = 2
_SC_SUBCORES = 16


def _enc_argmin_call(x, W1, b1, W2, b2, codebook):
    B, Din = x.shape
    H = W1.shape[1]
    K, D = codebook.shape
    NB = B // _BM
    NKC = K // _KN

    def body(x_ref, W1_ref, b1_ref, W2_ref, b2_ref, C_ref, z_ref, idx_ref,
             cn_ref, cm2_ref, iota_ref):
        i = pl.program_id(0)

        @pl.when(i == 0)
        def _():
            c = C_ref[...]
            cn_ref[...] = jnp.sum(c * c, axis=1, keepdims=True)
            cm2_ref[...] = c * -2.0
            iota_ref[...] = lax.broadcasted_iota(
                jnp.int32, (K, 1), 0).astype(jnp.float32)

        xb = x_ref[...]
        h = jnp.maximum(
            jnp.dot(xb, W1_ref[...], precision=lax.Precision.DEFAULT)
            + b1_ref[...], 0.0)
        z = jnp.dot(h, W2_ref[...],
                    precision=lax.Precision.DEFAULT) + b2_ref[...]
        z_ref[...] = z
        s = lax.dot_general(jnp.ones((1, D), jnp.float32), z * z,
                            (((1,), (1,)), ((), ())),
                            precision=lax.Precision.HIGHEST)

        best_v = jnp.full((1, _BM), jnp.inf, jnp.float32)
        best_i = jnp.full((1, _BM), jnp.inf, jnp.float32)
        for kc in range(NKC):
            Cc = cm2_ref[kc * _KN:(kc + 1) * _KN, :]
            m2 = lax.dot_general(Cc, z, (((1,), (1,)), ((), ())),
                                 precision=lax.Precision.DEFAULT)
            d = (s + m2) + cn_ref[kc * _KN:(kc + 1) * _KN, :]
            dmin = jnp.min(d, axis=0, keepdims=True)
            gi = iota_ref[kc * _KN:(kc + 1) * _KN, :]
            li = jnp.min(jnp.where(d == dmin, gi, jnp.inf), axis=0,
                         keepdims=True)
            upd = dmin < best_v
            best_i = jnp.where(upd, li, best_i)
            best_v = jnp.minimum(best_v, dmin)
        idx_ref[0, :, :] = best_i.astype(jnp.int32)

    return pl.pallas_call(
        body,
        grid=(NB,),
        in_specs=[
            pl.BlockSpec((_BM, Din), lambda i: (i, 0)),
            pl.BlockSpec((Din, H), lambda i: (0, 0)),
            pl.BlockSpec((1, H), lambda i: (0, 0)),
            pl.BlockSpec((H, D), lambda i: (0, 0)),
            pl.BlockSpec((1, D), lambda i: (0, 0)),
            pl.BlockSpec((K, D), lambda i: (0, 0)),
        ],
        out_specs=[
            pl.BlockSpec((_BM, D), lambda i: (i, 0)),
            pl.BlockSpec((1, 1, _BM), lambda i: (i, 0, 0)),
        ],
        out_shape=[
            jax.ShapeDtypeStruct((B, D), jnp.float32),
            jax.ShapeDtypeStruct((NB, 1, _BM), jnp.int32),
        ],
        scratch_shapes=[pltpu.VMEM((K, 1), jnp.float32),
                        pltpu.VMEM((K, D), jnp.float32),
                        pltpu.VMEM((K, 1), jnp.float32)],
    )(x, W1, b1, W2, b2, codebook)


def _sc_gather(table, idx):
    K, D = table.shape
    B = idx.shape[0]
    nw = _SC_CORES * _SC_SUBCORES
    b_per_w = B // nw
    mesh = plsc.VectorSubcoreMesh(core_axis_name="c", subcore_axis_name="s")

    @functools.partial(
        pl.kernel,
        mesh=mesh,
        out_type=jax.ShapeDtypeStruct((B, D), jnp.float32),
        scratch_types=[
            pltpu.VMEM((b_per_w,), jnp.int32),
            pltpu.VMEM((b_per_w, D), jnp.float32),
            pltpu.SemaphoreType.DMA,
        ],
    )
    def gather(table_hbm, idx_hbm, out_hbm, idx_v, rows_v, sem):
        wid = lax.axis_index("s") * _SC_CORES + lax.axis_index("c")
        base = wid * b_per_w
        pltpu.sync_copy(idx_hbm.at[pl.ds(base, b_per_w)], idx_v)
        pltpu.async_copy(table_hbm.at[idx_v], rows_v, sem).wait()
        pltpu.sync_copy(rows_v, out_hbm.at[pl.ds(base, b_per_w)])

    return gather(table, idx)


def _decoder_call(z, zq, Wd1, bd1, Wd2, bd2):
    B, D = z.shape
    H = Wd1.shape[1]
    Dout = Wd2.shape[1]
    NB = B // _BM

    def body(z_ref, zq_ref, Wd1_ref, bd1_ref, Wd2_ref, bd2_ref, recon_ref,
             loss_ref, acc_ref, w1b_ref, w2b_ref):
        i = pl.program_id(0)

        @pl.when(i == 0)
        def _():
            w1b_ref[...] = Wd1_ref[...].astype(jnp.bfloat16)
            w2b_ref[...] = Wd2_ref[...].astype(jnp.bfloat16)

        z = z_ref[...]
        zq = zq_ref[...]
        zq_st = z + (zq - z)
        hd = jnp.maximum(
            jnp.dot(zq_st.astype(jnp.bfloat16), w1b_ref[...],
                    preferred_element_type=jnp.float32) + bd1_ref[...], 0.0)
        y = jnp.dot(hd.astype(jnp.bfloat16), w2b_ref[...],
                    preferred_element_type=jnp.float32) + bd2_ref[...]
        recon_ref[...] = jax.nn.sigmoid(y)
        part = jnp.sum((zq - z) ** 2)
        acc = jnp.where(i == 0, 0.0, acc_ref[0]) + part
        acc_ref[0] = acc

        @pl.when(i == NB - 1)
        def _():
            loss_ref[...] = jnp.full((1, 1), acc, jnp.float32)

    return pl.pallas_call(
        body,
        grid=(NB,),
        in_specs=[
            pl.BlockSpec((_BM, D), lambda i: (i, 0)),
            pl.BlockSpec((_BM, D), lambda i: (i, 0)),
            pl.BlockSpec((D, H), lambda i: (0, 0)),
            pl.BlockSpec((1, H), lambda i: (0, 0)),
            pl.BlockSpec((H, Dout), lambda i: (0, 0)),
            pl.BlockSpec((1, Dout), lambda i: (0, 0)),
        ],
        out_specs=[
            pl.BlockSpec((_BM, Dout), lambda i: (i, 0)),
            pl.BlockSpec((1, 1), lambda i: (0, 0)),
        ],
        out_shape=[
            jax.ShapeDtypeStruct((B, Dout), jnp.float32),
            jax.ShapeDtypeStruct((1, 1), jnp.float32),
        ],
        scratch_shapes=[pltpu.SMEM((1,), jnp.float32),
                        pltpu.VMEM((D, H), jnp.bfloat16),
                        pltpu.VMEM((H, Dout), jnp.bfloat16)],
    )(z, zq, Wd1, bd1, Wd2, bd2)


def kernel(x, W1, b1, W2, b2, codebook, Wd1, bd1, Wd2, bd2):
    B = x.shape[0]
    z, idx3 = _enc_argmin_call(x, W1, b1.reshape(1, -1), W2,
                               b2.reshape(1, -1), codebook)
    indices = idx3.reshape(B)
    zq = _sc_gather(codebook, indices)
    recon, sum11 = _decoder_call(z, zq, Wd1, bd1.reshape(1, -1),
                                 Wd2, bd2.reshape(1, -1))
    mean = sum11.reshape(()) / (B * codebook.shape[1])
    loss = mean + 0.25 * mean
    return recon, loss, indices


# speedup vs baseline: 1.2160x; 1.0009x over previous
import functools

import jax
import jax.numpy as jnp
from jax import lax
from jax.experimental import pallas as pl
from jax.experimental.pallas import tpu as pltpu
from jax.experimental.pallas import tpu_sc as plsc

_BM = 1024
_KN = 1024
_SC_CORES = 2
_SC_SUBCORES = 16


def _enc_argmin_call(x, W1, b1, W2, b2, codebook):
    B, Din = x.shape
    H = W1.shape[1]
    K, D = codebook.shape
    NB = B // _BM
    NKC = K // _KN

    def body(x_ref, W1_ref, b1_ref, W2_ref, b2_ref, C_ref, z_ref, idx_ref,
             cn_ref, cm2_ref, iota_ref):
        i = pl.program_id(0)

        @pl.when(i == 0)
        def _():
            c = C_ref[...]
            cn_ref[...] = jnp.sum(c * c, axis=1, keepdims=True)
            cm2_ref[...] = c * -2.0
            iota_ref[...] = lax.broadcasted_iota(
                jnp.int32, (K, 1), 0).astype(jnp.float32)

        xb = x_ref[...]
        h = jnp.maximum(
            jnp.dot(xb, W1_ref[...], precision=lax.Precision.DEFAULT)
            + b1_ref[...], 0.0)
        z = jnp.dot(h, W2_ref[...],
                    precision=lax.Precision.DEFAULT) + b2_ref[...]
        z_ref[...] = z
        s = lax.dot_general(jnp.ones((1, D), jnp.float32), z * z,
                            (((1,), (1,)), ((), ())),
                            precision=lax.Precision.HIGHEST)

        best_v = jnp.full((1, _BM), jnp.inf, jnp.float32)
        best_i = jnp.full((1, _BM), jnp.inf, jnp.float32)
        for kc in range(NKC):
            Cc = cm2_ref[kc * _KN:(kc + 1) * _KN, :]
            m2 = lax.dot_general(Cc, z, (((1,), (1,)), ((), ())),
                                 precision=lax.Precision.DEFAULT)
            d = (s + m2) + cn_ref[kc * _KN:(kc + 1) * _KN, :]
            dmin = jnp.min(d, axis=0, keepdims=True)
            gi = iota_ref[kc * _KN:(kc + 1) * _KN, :]
            li = jnp.min(jnp.where(d == dmin, gi, jnp.inf), axis=0,
                         keepdims=True)
            upd = dmin < best_v
            best_i = jnp.where(upd, li, best_i)
            best_v = jnp.minimum(best_v, dmin)
        idx_ref[0, :, :] = best_i.astype(jnp.int32)

    return pl.pallas_call(
        body,
        grid=(NB,),
        in_specs=[
            pl.BlockSpec((_BM, Din), lambda i: (i, 0)),
            pl.BlockSpec((Din, H), lambda i: (0, 0)),
            pl.BlockSpec((1, H), lambda i: (0, 0)),
            pl.BlockSpec((H, D), lambda i: (0, 0)),
            pl.BlockSpec((1, D), lambda i: (0, 0)),
            pl.BlockSpec((K, D), lambda i: (0, 0)),
        ],
        out_specs=[
            pl.BlockSpec((_BM, D), lambda i: (i, 0)),
            pl.BlockSpec((1, 1, _BM), lambda i: (i, 0, 0)),
        ],
        out_shape=[
            jax.ShapeDtypeStruct((B, D), jnp.float32),
            jax.ShapeDtypeStruct((NB, 1, _BM), jnp.int32),
        ],
        scratch_shapes=[pltpu.VMEM((K, 1), jnp.float32),
                        pltpu.VMEM((K, D), jnp.float32),
                        pltpu.VMEM((K, 1), jnp.float32)],
    )(x, W1, b1, W2, b2, codebook)


def _sc_gather(table, idx):
    K, D = table.shape
    B = idx.shape[0]
    nw = _SC_CORES * _SC_SUBCORES
    b_per_w = B // nw
    mesh = plsc.VectorSubcoreMesh(core_axis_name="c", subcore_axis_name="s")

    @functools.partial(
        pl.kernel,
        mesh=mesh,
        out_type=jax.ShapeDtypeStruct((B, D), jnp.float32),
        scratch_types=[
            pltpu.VMEM((b_per_w,), jnp.int32),
            pltpu.VMEM((b_per_w, D), jnp.float32),
            pltpu.SemaphoreType.DMA,
        ],
    )
    def gather(table_hbm, idx_hbm, out_hbm, idx_v, rows_v, sem):
        wid = lax.axis_index("s") * _SC_CORES + lax.axis_index("c")
        base = wid * b_per_w
        pltpu.sync_copy(idx_hbm.at[pl.ds(base, b_per_w)], idx_v)
        pltpu.async_copy(table_hbm.at[idx_v], rows_v, sem).wait()
        pltpu.sync_copy(rows_v, out_hbm.at[pl.ds(base, b_per_w)])

    return gather(table, idx)


def _decoder_call(z, zq, Wd1, bd1, Wd2, bd2):
    B, D = z.shape
    H = Wd1.shape[1]
    Dout = Wd2.shape[1]
    NB = B // _BM

    def body(z_ref, zq_ref, Wd1_ref, bd1_ref, Wd2_ref, bd2_ref, recon_ref,
             loss_ref, acc_ref, w1b_ref, w2b_ref):
        i = pl.program_id(0)

        @pl.when(i == 0)
        def _():
            w1b_ref[...] = Wd1_ref[...].astype(jnp.bfloat16)
            w2b_ref[...] = Wd2_ref[...].astype(jnp.bfloat16)

        z = z_ref[...]
        zq = zq_ref[...]
        zq_st = z + (zq - z)
        hd = jnp.maximum(
            jnp.dot(zq_st.astype(jnp.bfloat16), w1b_ref[...],
                    preferred_element_type=jnp.float32) + bd1_ref[...], 0.0)
        y = jnp.dot(hd.astype(jnp.bfloat16), w2b_ref[...],
                    preferred_element_type=jnp.float32) + bd2_ref[...]
        recon_ref[...] = jax.nn.sigmoid(y)
        part = jnp.sum((zq - z) ** 2)
        acc = jnp.where(i == 0, 0.0, acc_ref[0]) + part
        acc_ref[0] = acc

        @pl.when(i == NB - 1)
        def _():
            loss_ref[...] = jnp.full((1, 1), acc, jnp.float32)

    return pl.pallas_call(
        body,
        grid=(NB,),
        in_specs=[
            pl.BlockSpec((_BM, D), lambda i: (i, 0)),
            pl.BlockSpec((_BM, D), lambda i: (i, 0)),
            pl.BlockSpec((D, H), lambda i: (0, 0)),
            pl.BlockSpec((1, H), lambda i: (0, 0)),
            pl.BlockSpec((H, Dout), lambda i: (0, 0)),
            pl.BlockSpec((1, Dout), lambda i: (0, 0)),
        ],
        out_specs=[
            pl.BlockSpec((_BM, Dout), lambda i: (i, 0)),
            pl.BlockSpec((1, 1), lambda i: (0, 0)),
        ],
        out_shape=[
            jax.ShapeDtypeStruct((B, Dout), jnp.float32),
            jax.ShapeDtypeStruct((1, 1), jnp.float32),
        ],
        scratch_shapes=[pltpu.SMEM((1,), jnp.float32),
                        pltpu.VMEM((D, H), jnp.bfloat16),
                        pltpu.VMEM((H, Dout), jnp.bfloat16)],
    )(z, zq, Wd1, bd1, Wd2, bd2)


def kernel(x, W1, b1, W2, b2, codebook, Wd1, bd1, Wd2, bd2):
    B = x.shape[0]
    z, idx3 = _enc_argmin_call(x, W1, b1.reshape(1, -1), W2,
                               b2.reshape(1, -1), codebook)
    indices = idx3.reshape(B)
    zq = _sc_gather(codebook, indices)
    recon, sum11 = _decoder_call(z, zq, Wd1, bd1.reshape(1, -1),
                                 Wd2, bd2.reshape(1, -1))
    mean = sum11.reshape(()) / (B * codebook.shape[1])
    loss = mean + 0.25 * mean
    return recon, loss, indices
